# baseline (speedup 1.0000x reference)
import functools

import jax
import jax.numpy as jnp
from jax import lax
from jax.experimental import pallas as pl
from jax.experimental.pallas import tpu as pltpu
from jax.experimental.pallas import tpu_sc as plsc

N = 10000
E = 320000
D = 128
EPS = 1e-5

NC = 2
NS = 16
NW = NC * NS
CHUNK = 120
CH_PER_W = 86
E_PAD = NW * CH_PER_W * CHUNK
STRIDE = 632
N_PAD = NS * STRIDE

_MESH = plsc.VectorSubcoreMesh(core_axis_name="c", subcore_axis_name="s")


def _sc_body(with_deg, *refs):
    if with_deg:
        (h_hbm, src_hbm, dst_hbm, zrow_hbm, zdeg_hbm, ones_hbm,
         agg_out, deg_out, agg_s, deg_s,
         src0, src1, src2, dst0, dst1, dst2,
         rows0, rows1, rows2, ones_v, deg_v,
         semi0, semi1, semi2, semg0, semg1, semg2,
         sems0, sems1, sems2) = refs
    else:
        (h_hbm, src_hbm, dst_hbm, zrow_hbm,
         agg_out, agg_s,
         src0, src1, src2, dst0, dst1, dst2,
         rows0, rows1, rows2,
         semi0, semi1, semi2, semg0, semg1, semg2,
         sems0, sems1, sems2) = refs
    srcv = (src0, src1, src2)
    dstv = (dst0, dst1, dst2)
    rowsb = (rows0, rows1, rows2)
    semi = (semi0, semi1, semi2)
    semg = (semg0, semg1, semg2)
    sems = (sems0, sems1, sems2)

    c = lax.axis_index("c")
    s = lax.axis_index("s")
    w = s * NC + c

    pltpu.sync_copy(zrow_hbm, agg_s.at[pl.ds(s * STRIDE, STRIDE)])
    if with_deg:
        pltpu.sync_copy(zdeg_hbm, deg_v)
        pltpu.sync_copy(deg_v, deg_s.at[pl.ds(s * STRIDE, STRIDE)])
        pltpu.sync_copy(ones_hbm, ones_v)
    plsc.subcore_barrier()

    def start_idx(k, b):
        off = (w * CH_PER_W + k) * CHUNK
        pltpu.async_copy(src_hbm.at[pl.ds(off, CHUNK)], srcv[b], semi[b])
        pltpu.async_copy(dst_hbm.at[pl.ds(off, CHUNK)], dstv[b], semi[b])

    def wait_idx(b):
        pltpu.make_async_copy(src_hbm.at[pl.ds(0, CHUNK)], srcv[b],
                              semi[b]).wait()
        pltpu.make_async_copy(dst_hbm.at[pl.ds(0, CHUNK)], dstv[b],
                              semi[b]).wait()

    def start_gather(b):
        pltpu.async_copy(h_hbm.at[srcv[b]], rowsb[b], semg[b])

    def wait_gather(b):
        pltpu.make_async_copy(h_hbm.at[srcv[b]], rowsb[b], semg[b]).wait()

    def start_scatter(b):
        pltpu.async_copy(rowsb[b], agg_s.at[dstv[b]], sems[b], add=True)
        if with_deg:
            pltpu.async_copy(ones_v, deg_s.at[dstv[b]], sems[b], add=True)

    def wait_scatter(b):
        pltpu.make_async_copy(rowsb[b], agg_s.at[dstv[b]], sems[b]).wait()
        if with_deg:
            pltpu.make_async_copy(ones_v, deg_s.at[dstv[b]], sems[b]).wait()

    def steady(k, b, first=False, last=False):
        bp = (b + 2) % 3
        bn = (b + 1) % 3
        wait_idx(b)
        start_gather(b)
        wait_gather(bp)
        start_scatter(bp)
        if not first:
            wait_scatter(bn)
        if not last:
            start_idx(k + 1, bn)

    start_idx(0, 0)
    start_idx(1, 1)
    wait_idx(0)
    start_gather(0)
    steady(1, 1, first=True)

    for k in range(2, CH_PER_W - 1):
        steady(k, k % 3)
    steady(CH_PER_W - 1, (CH_PER_W - 1) % 3, last=True)
    wait_gather(1)
    start_scatter(1)
    wait_scatter(0)
    wait_scatter(1)

    plsc.subcore_barrier()

    pltpu.sync_copy(agg_s.at[pl.ds(s * STRIDE, STRIDE)],
                    agg_out.at[c, pl.ds(s * STRIDE, STRIDE)])
    if with_deg:
        pltpu.sync_copy(deg_s.at[pl.ds(s * STRIDE, STRIDE)], deg_v)
        pltpu.sync_copy(deg_v,
                        deg_out.at[pl.ds(c * N_PAD + s * STRIDE, STRIDE)])


_sc_agg_deg = pl.kernel(
    functools.partial(_sc_body, True),
    out_type=(jax.ShapeDtypeStruct((NC, N_PAD, D), jnp.float32),
              jax.ShapeDtypeStruct((NC * N_PAD,), jnp.float32)),
    mesh=_MESH,
    scratch_types=[
        pltpu.VMEM_SHARED((N_PAD, D), jnp.float32),
        pltpu.VMEM_SHARED((N_PAD,), jnp.float32),
        pltpu.VMEM((CHUNK,), jnp.int32),
        pltpu.VMEM((CHUNK,), jnp.int32),
        pltpu.VMEM((CHUNK,), jnp.int32),
        pltpu.VMEM((CHUNK,), jnp.int32),
        pltpu.VMEM((CHUNK,), jnp.int32),
        pltpu.VMEM((CHUNK,), jnp.int32),
        pltpu.VMEM((CHUNK, D), jnp.float32),
        pltpu.VMEM((CHUNK, D), jnp.float32),
        pltpu.VMEM((CHUNK, D), jnp.float32),
        pltpu.VMEM((CHUNK,), jnp.float32),
        pltpu.VMEM((STRIDE,), jnp.float32),
    ] + [pltpu.SemaphoreType.DMA] * 9,
)

_sc_agg = pl.kernel(
    functools.partial(_sc_body, False),
    out_type=jax.ShapeDtypeStruct((NC, N_PAD, D), jnp.float32),
    mesh=_MESH,
    scratch_types=[
        pltpu.VMEM_SHARED((N_PAD, D), jnp.float32),
        pltpu.VMEM((CHUNK,), jnp.int32),
        pltpu.VMEM((CHUNK,), jnp.int32),
        pltpu.VMEM((CHUNK,), jnp.int32),
        pltpu.VMEM((CHUNK,), jnp.int32),
        pltpu.VMEM((CHUNK,), jnp.int32),
        pltpu.VMEM((CHUNK,), jnp.int32),
        pltpu.VMEM((CHUNK, D), jnp.float32),
        pltpu.VMEM((CHUNK, D), jnp.float32),
        pltpu.VMEM((CHUNK, D), jnp.float32),
    ] + [pltpu.SemaphoreType.DMA] * 9,
)


def _neigh(a_ref, d_ref):
    deg = jnp.maximum(d_ref[:N] + d_ref[N_PAD:N_PAD + N], 1.0)
    return (a_ref[0, :N] + a_ref[1, :N]) / deg[:, None]


def _tc_mid_body(h_ref, a_ref, d_ref, ws_ref, wn_ref, b_ref, g_ref, be_ref,
                 o_ref):
    hn = _neigh(a_ref, d_ref)
    z = (jnp.dot(h_ref[...], ws_ref[...], preferred_element_type=jnp.float32)
         + jnp.dot(hn, wn_ref[...], preferred_element_type=jnp.float32)
         + b_ref[...][None, :])
    mu = jnp.mean(z, axis=0)
    var = jnp.mean(z * z, axis=0) - mu * mu
    zn = (z - mu[None, :]) * lax.rsqrt(var + EPS)[None, :]
    zn = zn * g_ref[...][None, :] + be_ref[...][None, :]
    o_ref[...] = jnp.maximum(zn, 0.0)


def _tc_final_body(h_ref, a_ref, d_ref, ws_ref, wn_ref, b_ref, o_ref):
    hn = _neigh(a_ref, d_ref)
    z = (jnp.dot(h_ref[...], ws_ref[...], preferred_element_type=jnp.float32)
         + jnp.dot(hn, wn_ref[...], preferred_element_type=jnp.float32)
         + b_ref[...][None, :])
    m = jnp.max(z, axis=1, keepdims=True)
    lse = jnp.log(jnp.sum(jnp.exp(z - m), axis=1, keepdims=True)) + m
    o_ref[...] = z - lse


def _tc_mid(h, aggp, degp, Ws, Wn, b, g, be):
    return pl.pallas_call(
        _tc_mid_body,
        out_shape=jax.ShapeDtypeStruct((N, D), jnp.float32),
    )(h, aggp, degp, Ws, Wn, b, g, be)


def _tc_final(h, aggp, degp, Ws, Wn, b):
    return pl.pallas_call(
        _tc_final_body,
        out_shape=jax.ShapeDtypeStruct((N, D), jnp.float32),
    )(h, aggp, degp, Ws, Wn, b)


def kernel(x, edge_index, Ws0, Wn0, b0, g0, be0, Ws1, Wn1, b1, g1, be1,
           Ws2, Wn2, b2):
    pad = E_PAD - E
    iota = jnp.arange(pad, dtype=jnp.int32)
    src = jnp.concatenate([edge_index[0], iota % N])
    dst = jnp.concatenate([edge_index[1], N + iota % (N_PAD - N)])
    zrow = jnp.zeros((STRIDE, D), jnp.float32)
    zdeg = jnp.zeros((STRIDE,), jnp.float32)
    ones = jnp.ones((CHUNK,), jnp.float32)

    aggp0, degp = _sc_agg_deg(x, src, dst, zrow, zdeg, ones)
    h1 = _tc_mid(x, aggp0, degp, Ws0, Wn0, b0, g0, be0)
    aggp1 = _sc_agg(h1, src, dst, zrow)
    h2 = _tc_mid(h1, aggp1, degp, Ws1, Wn1, b1, g1, be1)
    aggp2 = _sc_agg(h2, src, dst, zrow)
    return _tc_final(h2, aggp2, degp, Ws2, Wn2, b2)


# speedup vs baseline: 1.0050x; 1.0050x over previous
import functools

import jax
import jax.numpy as jnp
from jax import lax
from jax.experimental import pallas as pl
from jax.experimental.pallas import tpu as pltpu
from jax.experimental.pallas import tpu_sc as plsc

N = 10000
E = 320000
D = 128
EPS = 1e-5

NC = 2
NS = 16
NW = NC * NS
CHUNK = 120
CH_PER_W = 86
E_PAD = NW * CH_PER_W * CHUNK
STRIDE = 632
N_PAD = NS * STRIDE

_MESH = plsc.VectorSubcoreMesh(core_axis_name="c", subcore_axis_name="s")


def _sc_body(with_deg, *refs):
    if with_deg:
        (h_hbm, src_hbm, dst_hbm, zrow_hbm, zdeg_hbm, ones_hbm,
         agg_out, deg_out, agg_s, deg_s,
         src0, src1, src2, dst0, dst1, dst2,
         rows0, rows1, rows2, ones_v, deg_v,
         semi0, semi1, semi2, semg0, semg1, semg2,
         sems0, sems1, sems2) = refs
    else:
        (h_hbm, src_hbm, dst_hbm, zrow_hbm,
         agg_out, agg_s,
         src0, src1, src2, dst0, dst1, dst2,
         rows0, rows1, rows2,
         semi0, semi1, semi2, semg0, semg1, semg2,
         sems0, sems1, sems2) = refs
    srcv = (src0, src1, src2)
    dstv = (dst0, dst1, dst2)
    rowsb = (rows0, rows1, rows2)
    semi = (semi0, semi1, semi2)
    semg = (semg0, semg1, semg2)
    sems = (sems0, sems1, sems2)

    c = lax.axis_index("c")
    s = lax.axis_index("s")
    w = s * NC + c

    pltpu.sync_copy(zrow_hbm, agg_s.at[pl.ds(s * STRIDE, STRIDE)])
    if with_deg:
        pltpu.sync_copy(zdeg_hbm, deg_v)
        pltpu.sync_copy(deg_v, deg_s.at[pl.ds(s * STRIDE, STRIDE)])
        pltpu.sync_copy(ones_hbm, ones_v)
    plsc.subcore_barrier()

    def start_idx(k, b):
        off = (w * CH_PER_W + k) * CHUNK
        pltpu.async_copy(src_hbm.at[pl.ds(off, CHUNK)], srcv[b], semi[b])
        pltpu.async_copy(dst_hbm.at[pl.ds(off, CHUNK)], dstv[b], semi[b])

    def wait_idx(b):
        pltpu.make_async_copy(src_hbm.at[pl.ds(0, CHUNK)], srcv[b],
                              semi[b]).wait()
        pltpu.make_async_copy(dst_hbm.at[pl.ds(0, CHUNK)], dstv[b],
                              semi[b]).wait()

    def start_gather(b):
        pltpu.async_copy(h_hbm.at[srcv[b]], rowsb[b], semg[b])

    def wait_gather(b):
        pltpu.make_async_copy(h_hbm.at[srcv[b]], rowsb[b], semg[b]).wait()

    def start_scatter(b):
        pltpu.async_copy(rowsb[b], agg_s.at[dstv[b]], sems[b], add=True)
        if with_deg:
            pltpu.async_copy(ones_v, deg_s.at[dstv[b]], sems[b], add=True)

    def wait_scatter(b):
        pltpu.make_async_copy(rowsb[b], agg_s.at[dstv[b]], sems[b]).wait()
        if with_deg:
            pltpu.make_async_copy(ones_v, deg_s.at[dstv[b]], sems[b]).wait()

    def steady(k, b, first=False, last=False):
        bp = (b + 2) % 3
        bn = (b + 1) % 3
        wait_idx(b)
        start_gather(b)
        wait_gather(bp)
        start_scatter(bp)
        if not first:
            wait_scatter(bn)
        if not last:
            start_idx(k + 1, bn)

    start_idx(0, 0)
    start_idx(1, 1)
    wait_idx(0)
    start_gather(0)
    steady(1, 1, first=True)

    def triple_body(j, carry):
        steady(3 * j + 2, 2)
        steady(3 * j + 3, 0)
        steady(3 * j + 4, 1)
        return carry

    lax.fori_loop(0, (CH_PER_W - 2) // 3 - 1, triple_body, 0)
    steady(CH_PER_W - 3, 2)
    steady(CH_PER_W - 2, 0)
    steady(CH_PER_W - 1, 1, last=True)
    wait_gather(1)
    start_scatter(1)
    wait_scatter(0)
    wait_scatter(1)

    plsc.subcore_barrier()

    pltpu.sync_copy(agg_s.at[pl.ds(s * STRIDE, STRIDE)],
                    agg_out.at[c, pl.ds(s * STRIDE, STRIDE)])
    if with_deg:
        pltpu.sync_copy(deg_s.at[pl.ds(s * STRIDE, STRIDE)], deg_v)
        pltpu.sync_copy(deg_v,
                        deg_out.at[pl.ds(c * N_PAD + s * STRIDE, STRIDE)])


_sc_agg_deg = pl.kernel(
    functools.partial(_sc_body, True),
    out_type=(jax.ShapeDtypeStruct((NC, N_PAD, D), jnp.float32),
              jax.ShapeDtypeStruct((NC * N_PAD,), jnp.float32)),
    mesh=_MESH,
    scratch_types=[
        pltpu.VMEM_SHARED((N_PAD, D), jnp.float32),
        pltpu.VMEM_SHARED((N_PAD,), jnp.float32),
        pltpu.VMEM((CHUNK,), jnp.int32),
        pltpu.VMEM((CHUNK,), jnp.int32),
        pltpu.VMEM((CHUNK,), jnp.int32),
        pltpu.VMEM((CHUNK,), jnp.int32),
        pltpu.VMEM((CHUNK,), jnp.int32),
        pltpu.VMEM((CHUNK,), jnp.int32),
        pltpu.VMEM((CHUNK, D), jnp.float32),
        pltpu.VMEM((CHUNK, D), jnp.float32),
        pltpu.VMEM((CHUNK, D), jnp.float32),
        pltpu.VMEM((CHUNK,), jnp.float32),
        pltpu.VMEM((STRIDE,), jnp.float32),
    ] + [pltpu.SemaphoreType.DMA] * 9,
)

_sc_agg = pl.kernel(
    functools.partial(_sc_body, False),
    out_type=jax.ShapeDtypeStruct((NC, N_PAD, D), jnp.float32),
    mesh=_MESH,
    scratch_types=[
        pltpu.VMEM_SHARED((N_PAD, D), jnp.float32),
        pltpu.VMEM((CHUNK,), jnp.int32),
        pltpu.VMEM((CHUNK,), jnp.int32),
        pltpu.VMEM((CHUNK,), jnp.int32),
        pltpu.VMEM((CHUNK,), jnp.int32),
        pltpu.VMEM((CHUNK,), jnp.int32),
        pltpu.VMEM((CHUNK,), jnp.int32),
        pltpu.VMEM((CHUNK, D), jnp.float32),
        pltpu.VMEM((CHUNK, D), jnp.float32),
        pltpu.VMEM((CHUNK, D), jnp.float32),
    ] + [pltpu.SemaphoreType.DMA] * 9,
)


def _neigh(a_ref, d_ref):
    deg = jnp.maximum(d_ref[:N] + d_ref[N_PAD:N_PAD + N], 1.0)
    return (a_ref[0, :N] + a_ref[1, :N]) / deg[:, None]


def _tc_self_body(h_ref, ws_ref, b_ref, o_ref):
    o_ref[...] = (jnp.dot(h_ref[...], ws_ref[...],
                          preferred_element_type=jnp.float32)
                  + b_ref[...][None, :])


def _tc_mid_body(self_ref, a_ref, d_ref, wn_ref, g_ref, be_ref, o_ref):
    hn = _neigh(a_ref, d_ref)
    z = (self_ref[...]
         + jnp.dot(hn, wn_ref[...], preferred_element_type=jnp.float32))
    mu = jnp.mean(z, axis=0)
    var = jnp.mean(z * z, axis=0) - mu * mu
    zn = (z - mu[None, :]) * lax.rsqrt(var + EPS)[None, :]
    zn = zn * g_ref[...][None, :] + be_ref[...][None, :]
    o_ref[...] = jnp.maximum(zn, 0.0)


def _tc_final_body(self_ref, a_ref, d_ref, wn_ref, o_ref):
    hn = _neigh(a_ref, d_ref)
    z = (self_ref[...]
         + jnp.dot(hn, wn_ref[...], preferred_element_type=jnp.float32))
    m = jnp.max(z, axis=1, keepdims=True)
    lse = jnp.log(jnp.sum(jnp.exp(z - m), axis=1, keepdims=True)) + m
    o_ref[...] = z - lse


def _tc_self(h, Ws, b):
    return pl.pallas_call(
        _tc_self_body,
        out_shape=jax.ShapeDtypeStruct((N, D), jnp.float32),
    )(h, Ws, b)


def _tc_mid(z0, aggp, degp, Wn, g, be):
    return pl.pallas_call(
        _tc_mid_body,
        out_shape=jax.ShapeDtypeStruct((N, D), jnp.float32),
    )(z0, aggp, degp, Wn, g, be)


def _tc_final(z0, aggp, degp, Wn):
    return pl.pallas_call(
        _tc_final_body,
        out_shape=jax.ShapeDtypeStruct((N, D), jnp.float32),
    )(z0, aggp, degp, Wn)


def kernel(x, edge_index, Ws0, Wn0, b0, g0, be0, Ws1, Wn1, b1, g1, be1,
           Ws2, Wn2, b2):
    pad = E_PAD - E
    iota = jnp.arange(pad, dtype=jnp.int32)
    src = jnp.concatenate([edge_index[0], iota % N])
    dst = jnp.concatenate([edge_index[1], N + iota % (N_PAD - N)])
    zrow = jnp.zeros((STRIDE, D), jnp.float32)
    zdeg = jnp.zeros((STRIDE,), jnp.float32)
    ones = jnp.ones((CHUNK,), jnp.float32)

    aggp0, degp = _sc_agg_deg(x, src, dst, zrow, zdeg, ones)
    s0 = _tc_self(x, Ws0, b0)
    h1 = _tc_mid(s0, aggp0, degp, Wn0, g0, be0)
    aggp1 = _sc_agg(h1, src, dst, zrow)
    s1 = _tc_self(h1, Ws1, b1)
    h2 = _tc_mid(s1, aggp1, degp, Wn1, g1, be1)
    aggp2 = _sc_agg(h2, src, dst, zrow)
    s2 = _tc_self(h2, Ws2, b2)
    return _tc_final(s2, aggp2, degp, Wn2)


# speedup vs baseline: 1.0173x; 1.0123x over previous
import functools

import jax
import jax.numpy as jnp
from jax import lax
from jax.experimental import pallas as pl
from jax.experimental.pallas import tpu as pltpu
from jax.experimental.pallas import tpu_sc as plsc

N = 10000
E = 320000
D = 128
EPS = 1e-5

NC = 2
NS = 16
NW = NC * NS
CHUNK = 120
CH_PER_W = 86
E_PAD = NW * CH_PER_W * CHUNK
STRIDE = 632
N_PAD = NS * STRIDE

_MESH = plsc.VectorSubcoreMesh(core_axis_name="c", subcore_axis_name="s")


def _sc_body(with_deg, *refs):
    if with_deg:
        (h_hbm, src_hbm, dst_hbm, zrow_hbm, zdeg_hbm, ones_hbm,
         agg_out, deg_out, agg_s, deg_s,
         src0, src1, src2, dst0, dst1, dst2,
         rows0, rows1, rows2, ones_v, deg_v,
         semi0, semi1, semi2, semg0, semg1, semg2,
         sems0, sems1, sems2) = refs
    else:
        (h_hbm, src_hbm, dst_hbm, zrow_hbm,
         agg_out, agg_s,
         src0, src1, src2, dst0, dst1, dst2,
         rows0, rows1, rows2,
         semi0, semi1, semi2, semg0, semg1, semg2,
         sems0, sems1, sems2) = refs
    srcv = (src0, src1, src2)
    dstv = (dst0, dst1, dst2)
    rowsb = (rows0, rows1, rows2)
    semi = (semi0, semi1, semi2)
    semg = (semg0, semg1, semg2)
    sems = (sems0, sems1, sems2)

    c = lax.axis_index("c")
    s = lax.axis_index("s")
    w = s * NC + c

    pltpu.sync_copy(zrow_hbm, agg_s.at[pl.ds(s * STRIDE, STRIDE)])
    if with_deg:
        pltpu.sync_copy(zdeg_hbm, deg_v)
        pltpu.sync_copy(deg_v, deg_s.at[pl.ds(s * STRIDE, STRIDE)])
        pltpu.sync_copy(ones_hbm, ones_v)
    plsc.subcore_barrier()

    def start_idx(k, b):
        off = (w * CH_PER_W + k) * CHUNK
        pltpu.async_copy(src_hbm.at[pl.ds(off, CHUNK)], srcv[b], semi[b])
        pltpu.async_copy(dst_hbm.at[pl.ds(off, CHUNK)], dstv[b], semi[b])

    def wait_idx(b):
        pltpu.make_async_copy(src_hbm.at[pl.ds(0, CHUNK)], srcv[b],
                              semi[b]).wait()
        pltpu.make_async_copy(dst_hbm.at[pl.ds(0, CHUNK)], dstv[b],
                              semi[b]).wait()

    def start_gather(b):
        pltpu.async_copy(h_hbm.at[srcv[b]], rowsb[b], semg[b])

    def wait_gather(b):
        pltpu.make_async_copy(h_hbm.at[srcv[b]], rowsb[b], semg[b]).wait()

    def start_scatter(b):
        pltpu.async_copy(rowsb[b], agg_s.at[dstv[b]], sems[b], add=True)
        if with_deg:
            pltpu.async_copy(ones_v, deg_s.at[dstv[b]], sems[b], add=True)

    def wait_scatter(b):
        pltpu.make_async_copy(rowsb[b], agg_s.at[dstv[b]], sems[b]).wait()
        if with_deg:
            pltpu.make_async_copy(ones_v, deg_s.at[dstv[b]], sems[b]).wait()

    def steady(k, b, first=False, last=False):
        bp = (b + 2) % 3
        bn = (b + 1) % 3
        wait_idx(b)
        start_gather(b)
        wait_gather(bp)
        start_scatter(bp)
        if not first:
            wait_scatter(bn)
        if not last:
            start_idx(k + 1, bn)

    start_idx(0, 0)
    start_idx(1, 1)
    wait_idx(0)
    start_gather(0)
    steady(1, 1, first=True)

    def triple_body(j, carry):
        steady(3 * j + 2, 2)
        steady(3 * j + 3, 0)
        steady(3 * j + 4, 1)
        return carry

    lax.fori_loop(0, (CH_PER_W - 2) // 3 - 1, triple_body, 0)
    steady(CH_PER_W - 3, 2)
    steady(CH_PER_W - 2, 0)
    steady(CH_PER_W - 1, 1, last=True)
    wait_gather(1)
    start_scatter(1)
    wait_scatter(0)
    wait_scatter(1)

    plsc.subcore_barrier()

    pltpu.sync_copy(agg_s.at[pl.ds(s * STRIDE, STRIDE)],
                    agg_out.at[c, pl.ds(s * STRIDE, STRIDE)])
    if with_deg:
        pltpu.sync_copy(deg_s.at[pl.ds(s * STRIDE, STRIDE)], deg_v)
        pltpu.sync_copy(deg_v,
                        deg_out.at[pl.ds(c * N_PAD + s * STRIDE, STRIDE)])


_sc_agg_deg = pl.kernel(
    functools.partial(_sc_body, True),
    out_type=(jax.ShapeDtypeStruct((NC, N_PAD, D), jnp.float32),
              jax.ShapeDtypeStruct((NC * N_PAD,), jnp.float32)),
    mesh=_MESH,
    scratch_types=[
        pltpu.VMEM_SHARED((N_PAD, D), jnp.float32),
        pltpu.VMEM_SHARED((N_PAD,), jnp.float32),
        pltpu.VMEM((CHUNK,), jnp.int32),
        pltpu.VMEM((CHUNK,), jnp.int32),
        pltpu.VMEM((CHUNK,), jnp.int32),
        pltpu.VMEM((CHUNK,), jnp.int32),
        pltpu.VMEM((CHUNK,), jnp.int32),
        pltpu.VMEM((CHUNK,), jnp.int32),
        pltpu.VMEM((CHUNK, D), jnp.float32),
        pltpu.VMEM((CHUNK, D), jnp.float32),
        pltpu.VMEM((CHUNK, D), jnp.float32),
        pltpu.VMEM((CHUNK,), jnp.float32),
        pltpu.VMEM((STRIDE,), jnp.float32),
    ] + [pltpu.SemaphoreType.DMA] * 9,
)

_sc_agg = pl.kernel(
    functools.partial(_sc_body, False),
    out_type=jax.ShapeDtypeStruct((NC, N_PAD, D), jnp.float32),
    mesh=_MESH,
    scratch_types=[
        pltpu.VMEM_SHARED((N_PAD, D), jnp.float32),
        pltpu.VMEM((CHUNK,), jnp.int32),
        pltpu.VMEM((CHUNK,), jnp.int32),
        pltpu.VMEM((CHUNK,), jnp.int32),
        pltpu.VMEM((CHUNK,), jnp.int32),
        pltpu.VMEM((CHUNK,), jnp.int32),
        pltpu.VMEM((CHUNK,), jnp.int32),
        pltpu.VMEM((CHUNK, D), jnp.float32),
        pltpu.VMEM((CHUNK, D), jnp.float32),
        pltpu.VMEM((CHUNK, D), jnp.float32),
    ] + [pltpu.SemaphoreType.DMA] * 9,
)


def _neigh(a_ref, d_ref):
    deg = jnp.maximum(d_ref[:N] + d_ref[N_PAD:N_PAD + N], 1.0)
    return (a_ref[0, :N] + a_ref[1, :N]) / deg[:, None]


def _tc_mid_body(h_ref, a_ref, d_ref, ws_ref, wn_ref, b_ref, g_ref, be_ref,
                 o_ref):
    hn = _neigh(a_ref, d_ref)
    z = (jnp.dot(h_ref[...], ws_ref[...], preferred_element_type=jnp.float32)
         + jnp.dot(hn, wn_ref[...], preferred_element_type=jnp.float32)
         + b_ref[...][None, :])
    mu = jnp.mean(z, axis=0)
    var = jnp.mean(z * z, axis=0) - mu * mu
    zn = (z - mu[None, :]) * lax.rsqrt(var + EPS)[None, :]
    zn = zn * g_ref[...][None, :] + be_ref[...][None, :]
    o_ref[...] = jnp.maximum(zn, 0.0)


def _tc_final_body(h_ref, a_ref, d_ref, ws_ref, wn_ref, b_ref, o_ref):
    hn = _neigh(a_ref, d_ref)
    z = (jnp.dot(h_ref[...], ws_ref[...], preferred_element_type=jnp.float32)
         + jnp.dot(hn, wn_ref[...], preferred_element_type=jnp.float32)
         + b_ref[...][None, :])
    m = jnp.max(z, axis=1, keepdims=True)
    lse = jnp.log(jnp.sum(jnp.exp(z - m), axis=1, keepdims=True)) + m
    o_ref[...] = z - lse


def _tc_mid(h, aggp, degp, Ws, Wn, b, g, be):
    return pl.pallas_call(
        _tc_mid_body,
        out_shape=jax.ShapeDtypeStruct((N, D), jnp.float32),
    )(h, aggp, degp, Ws, Wn, b, g, be)


def _tc_final(h, aggp, degp, Ws, Wn, b):
    return pl.pallas_call(
        _tc_final_body,
        out_shape=jax.ShapeDtypeStruct((N, D), jnp.float32),
    )(h, aggp, degp, Ws, Wn, b)


def kernel(x, edge_index, Ws0, Wn0, b0, g0, be0, Ws1, Wn1, b1, g1, be1,
           Ws2, Wn2, b2):
    pad = E_PAD - E
    iota = jnp.arange(pad, dtype=jnp.int32)
    src = jnp.concatenate([edge_index[0], iota % N])
    dst = jnp.concatenate([edge_index[1], N + iota % (N_PAD - N)])
    zrow = jnp.zeros((STRIDE, D), jnp.float32)
    zdeg = jnp.zeros((STRIDE,), jnp.float32)
    ones = jnp.ones((CHUNK,), jnp.float32)

    aggp0, degp = _sc_agg_deg(x, src, dst, zrow, zdeg, ones)
    h1 = _tc_mid(x, aggp0, degp, Ws0, Wn0, b0, g0, be0)
    aggp1 = _sc_agg(h1, src, dst, zrow)
    h2 = _tc_mid(h1, aggp1, degp, Ws1, Wn1, b1, g1, be1)
    aggp2 = _sc_agg(h2, src, dst, zrow)
    return _tc_final(h2, aggp2, degp, Ws2, Wn2, b2)


# speedup vs baseline: 1.0218x; 1.0043x over previous
import functools

import jax
import jax.numpy as jnp
from jax import lax
from jax.experimental import pallas as pl
from jax.experimental.pallas import tpu as pltpu
from jax.experimental.pallas import tpu_sc as plsc

N = 10000
E = 320000
D = 128
EPS = 1e-5

NC = 2
NS = 16
NW = NC * NS
CHUNK = 120
CH_PER_W = 86
E_PAD = NW * CH_PER_W * CHUNK
STRIDE = 632
N_PAD = NS * STRIDE

_MESH = plsc.VectorSubcoreMesh(core_axis_name="c", subcore_axis_name="s")


def _sc_body(with_deg, *refs):
    if with_deg:
        (h_hbm, src_hbm, dst_hbm, zrow_hbm, zdeg_hbm, ones_hbm,
         agg_out, deg_out, agg_s, deg_s,
         src0, src1, src2, dst0, dst1, dst2,
         rows0, rows1, rows2, ones_v, deg_v,
         semi0, semi1, semi2, semg0, semg1, semg2,
         sems0, sems1, sems2) = refs
    else:
        (h_hbm, src_hbm, dst_hbm, zrow_hbm,
         agg_out, agg_s,
         src0, src1, src2, dst0, dst1, dst2,
         rows0, rows1, rows2,
         semi0, semi1, semi2, semg0, semg1, semg2,
         sems0, sems1, sems2) = refs
    srcv = (src0, src1, src2)
    dstv = (dst0, dst1, dst2)
    rowsb = (rows0, rows1, rows2)
    semi = (semi0, semi1, semi2)
    semg = (semg0, semg1, semg2)
    sems = (sems0, sems1, sems2)

    c = lax.axis_index("c")
    s = lax.axis_index("s")
    w = s * NC + c

    pltpu.sync_copy(zrow_hbm.at[pl.ds(s * STRIDE, STRIDE)],
                    agg_s.at[pl.ds(s * STRIDE, STRIDE)])
    if with_deg:
        pltpu.sync_copy(zdeg_hbm.at[pl.ds(s * STRIDE, STRIDE)], deg_v)
        pltpu.sync_copy(deg_v, deg_s.at[pl.ds(s * STRIDE, STRIDE)])
        pltpu.sync_copy(ones_hbm, ones_v)
    plsc.subcore_barrier()

    def start_idx(k, b):
        off = (w * CH_PER_W + k) * CHUNK
        pltpu.async_copy(src_hbm.at[pl.ds(off, CHUNK)], srcv[b], semi[b])
        pltpu.async_copy(dst_hbm.at[pl.ds(off, CHUNK)], dstv[b], semi[b])

    def wait_idx(b):
        pltpu.make_async_copy(src_hbm.at[pl.ds(0, CHUNK)], srcv[b],
                              semi[b]).wait()
        pltpu.make_async_copy(dst_hbm.at[pl.ds(0, CHUNK)], dstv[b],
                              semi[b]).wait()

    def start_gather(b):
        pltpu.async_copy(h_hbm.at[srcv[b]], rowsb[b], semg[b])

    def wait_gather(b):
        pltpu.make_async_copy(h_hbm.at[srcv[b]], rowsb[b], semg[b]).wait()

    def start_scatter(b):
        pltpu.async_copy(rowsb[b], agg_s.at[dstv[b]], sems[b], add=True)
        if with_deg:
            pltpu.async_copy(ones_v, deg_s.at[dstv[b]], sems[b], add=True)

    def wait_scatter(b):
        pltpu.make_async_copy(rowsb[b], agg_s.at[dstv[b]], sems[b]).wait()
        if with_deg:
            pltpu.make_async_copy(ones_v, deg_s.at[dstv[b]], sems[b]).wait()

    def steady(k, b, first=False, last=False):
        bp = (b + 2) % 3
        bn = (b + 1) % 3
        wait_idx(b)
        start_gather(b)
        wait_gather(bp)
        start_scatter(bp)
        if not first:
            wait_scatter(bn)
        if not last:
            start_idx(k + 1, bn)

    start_idx(0, 0)
    start_idx(1, 1)
    wait_idx(0)
    start_gather(0)
    steady(1, 1, first=True)

    def triple_body(j, carry):
        steady(3 * j + 2, 2)
        steady(3 * j + 3, 0)
        steady(3 * j + 4, 1)
        return carry

    lax.fori_loop(0, (CH_PER_W - 2) // 3 - 1, triple_body, 0)
    steady(CH_PER_W - 3, 2)
    steady(CH_PER_W - 2, 0)
    steady(CH_PER_W - 1, 1, last=True)
    wait_gather(1)
    start_scatter(1)
    wait_scatter(0)
    wait_scatter(1)

    plsc.subcore_barrier()

    pltpu.sync_copy(agg_s.at[pl.ds(s * STRIDE, STRIDE)],
                    agg_out.at[c, pl.ds(s * STRIDE, STRIDE)])
    if with_deg:
        pltpu.sync_copy(deg_s.at[pl.ds(s * STRIDE, STRIDE)], deg_v)
        pltpu.sync_copy(deg_v,
                        deg_out.at[pl.ds(c * N_PAD + s * STRIDE, STRIDE)])


_sc_agg_deg = pl.kernel(
    functools.partial(_sc_body, True),
    out_type=(jax.ShapeDtypeStruct((NC, N_PAD, D), jnp.float32),
              jax.ShapeDtypeStruct((NC * N_PAD,), jnp.float32)),
    mesh=_MESH,
    scratch_types=[
        pltpu.VMEM_SHARED((N_PAD, D), jnp.float32),
        pltpu.VMEM_SHARED((N_PAD,), jnp.float32),
        pltpu.VMEM((CHUNK,), jnp.int32),
        pltpu.VMEM((CHUNK,), jnp.int32),
        pltpu.VMEM((CHUNK,), jnp.int32),
        pltpu.VMEM((CHUNK,), jnp.int32),
        pltpu.VMEM((CHUNK,), jnp.int32),
        pltpu.VMEM((CHUNK,), jnp.int32),
        pltpu.VMEM((CHUNK, D), jnp.float32),
        pltpu.VMEM((CHUNK, D), jnp.float32),
        pltpu.VMEM((CHUNK, D), jnp.float32),
        pltpu.VMEM((CHUNK,), jnp.float32),
        pltpu.VMEM((STRIDE,), jnp.float32),
    ] + [pltpu.SemaphoreType.DMA] * 9,
)

_sc_agg = pl.kernel(
    functools.partial(_sc_body, False),
    out_type=jax.ShapeDtypeStruct((NC, N_PAD, D), jnp.float32),
    mesh=_MESH,
    scratch_types=[
        pltpu.VMEM_SHARED((N_PAD, D), jnp.float32),
        pltpu.VMEM((CHUNK,), jnp.int32),
        pltpu.VMEM((CHUNK,), jnp.int32),
        pltpu.VMEM((CHUNK,), jnp.int32),
        pltpu.VMEM((CHUNK,), jnp.int32),
        pltpu.VMEM((CHUNK,), jnp.int32),
        pltpu.VMEM((CHUNK,), jnp.int32),
        pltpu.VMEM((CHUNK, D), jnp.float32),
        pltpu.VMEM((CHUNK, D), jnp.float32),
        pltpu.VMEM((CHUNK, D), jnp.float32),
    ] + [pltpu.SemaphoreType.DMA] * 9,
)


def _neigh(a_ref, d_ref):
    deg = jnp.maximum(d_ref[:N] + d_ref[N_PAD:N_PAD + N], 1.0)
    return (a_ref[0, :N] + a_ref[1, :N]) / deg[:, None]


def _tc_mid_body(h_ref, a_ref, d_ref, ws_ref, wn_ref, b_ref, g_ref, be_ref,
                 o_ref):
    hn = _neigh(a_ref, d_ref)
    z = (jnp.dot(h_ref[...], ws_ref[...], preferred_element_type=jnp.float32)
         + jnp.dot(hn, wn_ref[...], preferred_element_type=jnp.float32)
         + b_ref[...][None, :])
    mu = jnp.mean(z, axis=0)
    var = jnp.mean(z * z, axis=0) - mu * mu
    zn = (z - mu[None, :]) * lax.rsqrt(var + EPS)[None, :]
    zn = zn * g_ref[...][None, :] + be_ref[...][None, :]
    o_ref[...] = jnp.maximum(zn, 0.0)


def _tc_final_body(h_ref, a_ref, d_ref, ws_ref, wn_ref, b_ref, o_ref):
    hn = _neigh(a_ref, d_ref)
    z = (jnp.dot(h_ref[...], ws_ref[...], preferred_element_type=jnp.float32)
         + jnp.dot(hn, wn_ref[...], preferred_element_type=jnp.float32)
         + b_ref[...][None, :])
    m = jnp.max(z, axis=1, keepdims=True)
    lse = jnp.log(jnp.sum(jnp.exp(z - m), axis=1, keepdims=True)) + m
    o_ref[...] = z - lse


def _tc_mid(h, aggp, degp, Ws, Wn, b, g, be):
    return pl.pallas_call(
        _tc_mid_body,
        out_shape=jax.ShapeDtypeStruct((N, D), jnp.float32),
    )(h, aggp, degp, Ws, Wn, b, g, be)


def _tc_final(h, aggp, degp, Ws, Wn, b):
    return pl.pallas_call(
        _tc_final_body,
        out_shape=jax.ShapeDtypeStruct((N, D), jnp.float32),
    )(h, aggp, degp, Ws, Wn, b)


def kernel(x, edge_index, Ws0, Wn0, b0, g0, be0, Ws1, Wn1, b1, g1, be1,
           Ws2, Wn2, b2):
    pad = E_PAD - E
    iota = jnp.arange(pad, dtype=jnp.int32)
    src = jnp.concatenate([edge_index[0], iota % N])
    dst = jnp.concatenate([edge_index[1], N + iota % (N_PAD - N)])
    zrow = jnp.zeros((N_PAD, D), jnp.float32)
    zdeg = jnp.zeros((N_PAD,), jnp.float32)
    ones = jnp.ones((CHUNK,), jnp.float32)

    aggp0, degp = _sc_agg_deg(x, src, dst, zrow, zdeg, ones)
    h1 = _tc_mid(x, aggp0, degp, Ws0, Wn0, b0, g0, be0)
    aggp1 = _sc_agg(h1, src, dst, zrow)
    h2 = _tc_mid(h1, aggp1, degp, Ws1, Wn1, b1, g1, be1)
    aggp2 = _sc_agg(h2, src, dst, zrow)
    return _tc_final(h2, aggp2, degp, Ws2, Wn2, b2)
